# baseline (speedup 1.0000x reference)
import functools

import jax
import jax.numpy as jnp
import jax.scipy.linalg as jsl
from jax.experimental import pallas as pl


def _tokenizer_kernel(x_ref, y_ref, w_ref, t_ref, o_ref, *, bb):
    classes = jax.lax.broadcasted_iota(jnp.int32, (1, 128), 1)
    for jb in range(bb):
        xj = x_ref[jb]
        acc = jax.lax.dot_general(
            xj, w_ref[...],
            dimension_numbers=(((1,), (0,)), ((), ())),
            preferred_element_type=jnp.float32,
        )
        for j in range(4):
            yjs = y_ref[jb, 512 * j:512 * (j + 1)][:, None]
            onehot = (yjs == classes).astype(jnp.float32)
            acc += jax.lax.dot_general(
                onehot, t_ref[j],
                dimension_numbers=(((1,), (0,)), ((), ())),
                preferred_element_type=jnp.float32,
            )
        for j in range(4):
            o_ref[jb, 512 * j:512 * (j + 1), :] = acc[:, 64 * j:64 * (j + 1)]


@jax.jit
def kernel(x, y, W, b, table):
    B, N, D = x.shape
    H, _ = W.shape
    C = table.shape[0]
    BB = 8

    table_b = table + b[None, :]
    Wt = W.T
    Wd = jsl.block_diag(Wt, Wt, Wt, Wt)
    T4 = jnp.zeros((4, 128, 4 * H), jnp.float32)
    for j in range(4):
        T4 = T4.at[j, :C, j * H:(j + 1) * H].set(table_b)

    xp = jnp.transpose(x.reshape(B, 4, N // 4, D), (0, 2, 1, 3)).reshape(
        B, N // 4, 4 * D)

    out = pl.pallas_call(
        functools.partial(_tokenizer_kernel, bb=BB),
        grid=(B // BB,),
        in_specs=[
            pl.BlockSpec((BB, N // 4, 4 * D), lambda i: (i, 0, 0)),
            pl.BlockSpec((BB, N), lambda i: (i, 0)),
            pl.BlockSpec((4 * D, 4 * H), lambda i: (0, 0)),
            pl.BlockSpec((4, 128, 4 * H), lambda i: (0, 0, 0)),
        ],
        out_specs=pl.BlockSpec((BB, N, H), lambda i: (i, 0, 0)),
        out_shape=jax.ShapeDtypeStruct((B, N, H), jnp.float32),
    )(xp, y, Wd, T4)
    return out


# speedup vs baseline: 1.1806x; 1.1806x over previous
import functools

import jax
import jax.numpy as jnp
import jax.scipy.linalg as jsl
from jax.experimental import pallas as pl


def _tokenizer_kernel(x_ref, y_ref, w_ref, t_ref, o_ref, *, bb):
    classes = jax.lax.broadcasted_iota(jnp.int32, (1, 128), 1)
    for jb in range(bb):
        xj = x_ref[jb]
        acc = jax.lax.dot_general(
            xj, w_ref[...],
            dimension_numbers=(((1,), (0,)), ((), ())),
            preferred_element_type=jnp.float32,
        )
        for j in range(4):
            yjs = y_ref[jb, j][:, None]
            onehot = (yjs == classes).astype(jnp.float32)
            acc += jax.lax.dot_general(
                onehot, t_ref[j],
                dimension_numbers=(((1,), (0,)), ((), ())),
                preferred_element_type=jnp.float32,
            )
        o_ref[jb] = acc.reshape(1024, 128)


@jax.jit
def kernel(x, y, W, b, table):
    B, N, D = x.shape
    H, _ = W.shape
    C = table.shape[0]
    BB = 8

    table_b = table + b[None, :]
    Wt = W.T
    Wd = jsl.block_diag(Wt, Wt, Wt, Wt)
    T4 = jnp.zeros((4, 128, 4 * H), jnp.float32)
    for j in range(4):
        T4 = T4.at[j, :C, j * H:(j + 1) * H].set(table_b)

    call = pl.pallas_call(
        functools.partial(_tokenizer_kernel, bb=BB),
        grid=(B // BB,),
        in_specs=[
            pl.BlockSpec((BB, N // 4, 4 * D), lambda i: (i, 0, 0)),
            pl.BlockSpec((BB, 4, N // 4), lambda i: (i, 0, 0)),
            pl.BlockSpec((4 * D, 4 * H), lambda i: (0, 0)),
            pl.BlockSpec((4, 128, 4 * H), lambda i: (0, 0, 0)),
        ],
        out_specs=pl.BlockSpec((BB, N // 2, 2 * H), lambda i: (i, 0, 0)),
        out_shape=jax.ShapeDtypeStruct((B, N // 2, 2 * H), jnp.float32),
    )

    xp = x.reshape(B, N // 4, 4 * D)
    ys = jnp.transpose(y.reshape(B, N // 4, 4), (0, 2, 1))
    out = call(xp, ys, Wd, T4)
    return out.reshape(B, N, H)
